# baseline (speedup 1.0000x reference)
import functools

import jax
import jax.numpy as jnp
from jax import lax
from jax.experimental import pallas as pl
from jax.experimental.pallas import tpu as pltpu
from jax.experimental.pallas import tpu_sc as plsc

_N = 10000
_E = 320000
_D = 128
_DE = 16
_DU = 128

_NC = 2
_NS = 16
_NW = _NC * _NS
_NPAD = 10240
_PER = _NPAD // _NS

_NHALF = 2
_EH = _E // _NHALF
_SH = _EH // 8
_EPW = _EH // _NW
_CHUNK = 128
_NCH = _EPW // _CHUNK
_TAIL = _EPW - _NCH * _CHUNK
_NBUF = 3
_ABLK = 2000


def _edge_proj_body(e0, e1, e2, e3, e4, e5, e6, e7,
                    wea_ref, u_ref, weu_ref, be_ref, out_ref):
    c = jnp.dot(u_ref[...], weu_ref[...], preferred_element_type=jnp.float32)
    c = c + be_ref[...]
    w = wea_ref[...]
    parts = [
        jnp.dot(e[...], w, preferred_element_type=jnp.float32) + c
        for e in (e0, e1, e2, e3, e4, e5, e6, e7)
    ]
    out_ref[...] = jnp.concatenate(parts, axis=1)


def _make_edge_proj(half):
    nblk = _SH // _ABLK
    blk0 = half * (_EH // _ABLK)

    def make_map(j):
        return lambda i: (blk0 + i + j * nblk, 0)

    ea_specs = [pl.BlockSpec((_ABLK, _DE), make_map(j)) for j in range(8)]

    def call(edge_attr, we_a, u, we_u, b_e2):
        return pl.pallas_call(
            _edge_proj_body,
            grid=(nblk,),
            in_specs=ea_specs + [
                pl.BlockSpec((_DE, _DE), lambda i: (0, 0)),
                pl.BlockSpec((1, 128), lambda i: (0, 0)),
                pl.BlockSpec((128, _DE), lambda i: (0, 0)),
                pl.BlockSpec((1, _DE), lambda i: (0, 0)),
            ],
            out_specs=pl.BlockSpec((_ABLK, 128), lambda i: (i, 0)),
            out_shape=jax.ShapeDtypeStruct((_SH, 128), jnp.float32),
        )(*([edge_attr] * 8), we_a, u, we_u, b_e2)

    return call


_EDGE_PROJ = tuple(_make_edge_proj(h) for h in range(_NHALF))


def _node_proj_body(x_ref, ws_ref, wd_ref, ps_ref, pd_ref):
    x = x_ref[...]
    ps_ref[...] = jnp.dot(x, ws_ref[...], preferred_element_type=jnp.float32)
    pd_ref[...] = jnp.dot(x, wd_ref[...], preferred_element_type=jnp.float32)


def _node_proj(x, we_s, we_d):
    blk = 2000
    return pl.pallas_call(
        _node_proj_body,
        grid=(_N // blk,),
        in_specs=[
            pl.BlockSpec((blk, _D), lambda i: (i, 0)),
            pl.BlockSpec((_D, _DE), lambda i: (0, 0)),
            pl.BlockSpec((_D, _DE), lambda i: (0, 0)),
        ],
        out_specs=[
            pl.BlockSpec((blk, _DE), lambda i: (i, 0)),
            pl.BlockSpec((blk, _DE), lambda i: (i, 0)),
        ],
        out_shape=[
            jax.ShapeDtypeStruct((_N, _DE), jnp.float32),
            jax.ShapeDtypeStruct((_N, _DE), jnp.float32),
        ],
    )(x, we_s, we_d)


_MESH = plsc.VectorSubcoreMesh(
    core_axis_name="c", subcore_axis_name="s", num_cores=_NC, num_subcores=_NS
)


def _make_edge_sc(half):
    edge0 = half * _EH

    @functools.partial(
        pl.kernel,
        out_type=jax.ShapeDtypeStruct((2, _NPAD, _DE), jnp.float32),
        mesh=_MESH,
        scratch_types=(
            [pltpu.VMEM_SHARED((_NPAD, _DE), jnp.float32)]
            + [pltpu.VMEM((_CHUNK,), jnp.int32) for _ in range(2 * _NBUF)]
            + [pltpu.VMEM((_CHUNK, _DE), jnp.float32) for _ in range(3 * _NBUF)]
            + [pltpu.VMEM((_TAIL,), jnp.int32) for _ in range(2)]
            + [pltpu.VMEM((_TAIL, _DE), jnp.float32) for _ in range(3)]
            + [pltpu.SemaphoreType.DMA for _ in range(3 * _NBUF)]
        ),
        compiler_params=pltpu.CompilerParams(use_tc_tiling_on_sc=False),
        name=f"edge_sc_h{half}",
    )
    def edge_sc(a_hbm, src_hbm, dst_hbm, ps_hbm, pd_hbm, zero_hbm, out_hbm,
                ebar_sh,
                si0, si1, si2, di0, di1, di2,
                ar0, ar1, ar2, psr0, psr1, psr2, pdr0, pdr1, pdr2,
                tsi, tdi, tar, tps, tpd,
                lin0, lin1, lin2, g0, g1, g2, sc0, sc1, sc2):
        c = lax.axis_index("c")
        s = lax.axis_index("s")
        wid = s * _NC + c

        pltpu.sync_copy(zero_hbm.at[pl.ds(s * _PER, _PER)],
                        ebar_sh.at[pl.ds(s * _PER, _PER)])
        plsc.subcore_barrier()

        base0 = edge0 + wid * _EPW
        acol = (wid // 4) * _DE
        airow = (wid % 4) * _EPW
        si = (si0, si1, si2)
        di = (di0, di1, di2)
        ar = (ar0, ar1, ar2)
        psr = (psr0, psr1, psr2)
        pdr = (pdr0, pdr1, pdr2)
        lin = (lin0, lin1, lin2)
        gse = (g0, g1, g2)
        sce = (sc0, sc1, sc2)

        def lin_cps(j, b):
            base = base0 + j * _CHUNK
            return (
                pltpu.make_async_copy(src_hbm.at[pl.ds(base, _CHUNK)], si[b],
                                      lin[b]),
                pltpu.make_async_copy(dst_hbm.at[pl.ds(base, _CHUNK)], di[b],
                                      lin[b]),
                pltpu.make_async_copy(
                    a_hbm.at[pl.ds(airow + j * _CHUNK, _CHUNK),
                             pl.ds(acol, _DE)],
                    ar[b], lin[b]),
            )

        def g_cps(b):
            return (
                pltpu.make_async_copy(ps_hbm.at[si[b]], psr[b], gse[b]),
                pltpu.make_async_copy(pd_hbm.at[di[b]], pdr[b], gse[b]),
            )

        def issue_lin(j, b):
            for cp in lin_cps(j, b):
                cp.start()

        def wait_lin(j, b):
            for cp in lin_cps(j, b):
                cp.wait()

        def issue_g(b):
            for cp in g_cps(b):
                cp.start()

        def wait_g(b):
            for cp in g_cps(b):
                cp.wait()

        def issue_sc(b):
            pltpu.async_copy(ar[b], ebar_sh.at[di[b]], sce[b], add=True)

        def wait_sc(b):
            pltpu.make_async_copy(ar[b], ebar_sh.at[di[b]], sce[b]).wait()

        def compute(b):
            a, p, q = ar[b], psr[b], pdr[b]
            for r in range(_CHUNK):
                a[r] = jnp.maximum(a[r] + p[r] + q[r], 0.0)

        def step(j, b, b1, b2, g_next=True, lin_next=True, wait_prev=True):
            if g_next:
                wait_lin(j + 1, b1)
                issue_g(b1)
            wait_g(b)
            compute(b)
            if wait_prev:
                wait_sc(b2)
            issue_sc(b)
            if lin_next:
                issue_lin(j + 2, b2)

        issue_lin(0, 0)
        wait_lin(0, 0)
        issue_g(0)
        issue_lin(1, 1)
        step(0, 0, 1, 2, wait_prev=False)
        step(1, 1, 2, 0)
        step(2, 2, 0, 1)

        def group(jj, carry):
            j = 3 * jj
            step(j + 0, 0, 1, 2)
            step(j + 1, 1, 2, 0)
            step(j + 2, 2, 0, 1)
            return carry

        lax.fori_loop(1, _NCH // _NBUF - 1, group, 0)

        step(_NCH - 3, 0, 1, 2)
        step(_NCH - 2, 1, 2, 0, lin_next=False)
        step(_NCH - 1, 2, 0, 1, g_next=False, lin_next=False)
        wait_sc(2)

        tbase = base0 + _NCH * _CHUNK
        pltpu.sync_copy(src_hbm.at[pl.ds(tbase, _TAIL)], tsi)
        pltpu.sync_copy(dst_hbm.at[pl.ds(tbase, _TAIL)], tdi)
        pltpu.sync_copy(
            a_hbm.at[pl.ds(airow + _NCH * _CHUNK, _TAIL), pl.ds(acol, _DE)],
            tar)
        pltpu.async_copy(ps_hbm.at[tsi], tps, g0).wait()
        pltpu.async_copy(pd_hbm.at[tdi], tpd, g1).wait()
        for r in range(_TAIL):
            tar[r] = jnp.maximum(tar[r] + tps[r] + tpd[r], 0.0)
        pltpu.sync_copy(tar, ebar_sh.at[tdi], add=True)

        plsc.subcore_barrier()
        pltpu.sync_copy(ebar_sh.at[pl.ds(s * _PER, _PER)],
                        out_hbm.at[c].at[pl.ds(s * _PER, _PER)])

    return edge_sc


_EDGE_SC = tuple(_make_edge_sc(h) for h in range(_NHALF))


def _node_update_body(eb0_ref, eb1_ref, x_ref, u_ref, wve_ref, wvx_ref,
                      wvu_ref, bv_ref, wue_ref, wuv_ref, wuu_ref, bu_ref,
                      v_ref, uo_ref, esum, vsum):
    j = pl.program_id(0)
    nb = pl.num_programs(0)

    eb = eb0_ref[0] + eb0_ref[1] + eb1_ref[0] + eb1_ref[1]
    acc = jnp.dot(eb, wve_ref[...], preferred_element_type=jnp.float32)
    acc = acc + jnp.dot(x_ref[...], wvx_ref[...],
                        preferred_element_type=jnp.float32)
    acc = acc + jnp.dot(u_ref[...], wvu_ref[...],
                        preferred_element_type=jnp.float32)
    acc = acc + bv_ref[...]
    v = jnp.maximum(acc, 0.0)
    v_ref[...] = v

    @pl.when(j == 0)
    def _():
        esum[...] = jnp.zeros_like(esum)
        vsum[...] = jnp.zeros_like(vsum)

    esum[...] += jnp.sum(eb, axis=0, keepdims=True)
    vsum[...] += jnp.sum(v, axis=0, keepdims=True)

    @pl.when(j == nb - 1)
    def _():
        un = jnp.dot(esum[...] * (1.0 / _E), wue_ref[...],
                     preferred_element_type=jnp.float32)
        un = un + jnp.dot(vsum[...] * (1.0 / _N), wuv_ref[...],
                          preferred_element_type=jnp.float32)
        un = un + jnp.dot(u_ref[...], wuu_ref[...],
                          preferred_element_type=jnp.float32)
        uo_ref[...] = un + bu_ref[...]


def _node_update(eb0, eb1, x, u, wv_e, wv_x, wv_u, bv2, wu_e, wu_v, wu_u,
                 bu2):
    blk = 2000
    return pl.pallas_call(
        _node_update_body,
        grid=(_N // blk,),
        in_specs=[
            pl.BlockSpec((2, blk, _DE), lambda i: (0, i, 0)),
            pl.BlockSpec((2, blk, _DE), lambda i: (0, i, 0)),
            pl.BlockSpec((blk, _D), lambda i: (i, 0)),
            pl.BlockSpec((1, _DU), lambda i: (0, 0)),
            pl.BlockSpec((_DE, _D), lambda i: (0, 0)),
            pl.BlockSpec((_D, _D), lambda i: (0, 0)),
            pl.BlockSpec((_DU, _D), lambda i: (0, 0)),
            pl.BlockSpec((1, _D), lambda i: (0, 0)),
            pl.BlockSpec((_DE, _DU), lambda i: (0, 0)),
            pl.BlockSpec((_D, _DU), lambda i: (0, 0)),
            pl.BlockSpec((_DU, _DU), lambda i: (0, 0)),
            pl.BlockSpec((1, _DU), lambda i: (0, 0)),
        ],
        out_specs=[
            pl.BlockSpec((blk, _D), lambda i: (i, 0)),
            pl.BlockSpec((1, _DU), lambda i: (0, 0)),
        ],
        out_shape=[
            jax.ShapeDtypeStruct((_N, _D), jnp.float32),
            jax.ShapeDtypeStruct((1, _DU), jnp.float32),
        ],
        scratch_shapes=[
            pltpu.VMEM((1, _DE), jnp.float32),
            pltpu.VMEM((1, _D), jnp.float32),
        ],
    )(eb0, eb1, x, u, wv_e, wv_x, wv_u, bv2, wu_e, wu_v, wu_u, bu2)


def kernel(x, edge_index, edge_attr, u, W_e, b_e, W_v, b_v, W_u, b_u):
    x = x.astype(jnp.float32)
    src = edge_index[0].astype(jnp.int32)
    dst = edge_index[1].astype(jnp.int32)

    we_a = W_e[0:_DE]
    we_s = W_e[_DE:_DE + _D]
    we_d = W_e[_DE + _D:_DE + 2 * _D]
    we_u = W_e[_DE + 2 * _D:]

    ps, pd = _node_proj(x, we_s, we_d)
    zeros = jnp.zeros((_NPAD, _DE), jnp.float32)
    b_e2 = b_e.reshape(1, _DE)

    ebar = []
    for h in range(_NHALF):
        a_h = _EDGE_PROJ[h](edge_attr, we_a, u, we_u, b_e2)
        ebar.append(_EDGE_SC[h](a_h, src, dst, ps, pd, zeros))

    wv_e = W_v[0:_DE]
    wv_x = W_v[_DE:_DE + _D]
    wv_u = W_v[_DE + _D:]
    wu_e = W_u[0:_DE]
    wu_v = W_u[_DE:_DE + _D]
    wu_u = W_u[_DE + _D:]

    v_new, u_new = _node_update(
        ebar[0], ebar[1], x, u, wv_e, wv_x, wv_u, b_v.reshape(1, _D),
        wu_e, wu_v, wu_u, b_u.reshape(1, _DU))
    return v_new, u_new


# speedup vs baseline: 1.0579x; 1.0579x over previous
import functools

import jax
import jax.numpy as jnp
from jax import lax
from jax.experimental import pallas as pl
from jax.experimental.pallas import tpu as pltpu
from jax.experimental.pallas import tpu_sc as plsc

_N = 10000
_E = 320000
_D = 128
_DE = 16
_DU = 128

_NC = 2
_NS = 16
_NW = _NC * _NS
_NPAD = 10240
_PER = _NPAD // _NS

_NHALF = 2
_EH = _E // _NHALF
_SH = _EH // 8
_EPW = _EH // _NW
_CHUNK = 128
_NCH = _EPW // _CHUNK
_TAIL = _EPW - _NCH * _CHUNK
_NBUF = 3
_ABLK = 2000


def _edge_proj_body(e0, e1, e2, e3, e4, e5, e6, e7,
                    wea_ref, u_ref, weu_ref, be_ref, out_ref):
    c = jnp.dot(u_ref[...], weu_ref[...], preferred_element_type=jnp.float32)
    c = c + be_ref[...]
    w = wea_ref[...]
    parts = [
        jnp.dot(e[...], w, preferred_element_type=jnp.float32) + c
        for e in (e0, e1, e2, e3, e4, e5, e6, e7)
    ]
    out_ref[...] = jnp.concatenate(parts, axis=1)


def _make_edge_proj(half):
    del half
    nblk = _SH // _ABLK

    def make_map(j):
        return lambda i: (i + j * nblk, 0)

    ea_specs = [pl.BlockSpec((_ABLK, _DE), make_map(j)) for j in range(8)]

    def call(edge_attr, we_a, u, we_u, b_e2):
        return pl.pallas_call(
            _edge_proj_body,
            grid=(nblk,),
            in_specs=ea_specs + [
                pl.BlockSpec((_DE, _DE), lambda i: (0, 0)),
                pl.BlockSpec((1, 128), lambda i: (0, 0)),
                pl.BlockSpec((128, _DE), lambda i: (0, 0)),
                pl.BlockSpec((1, _DE), lambda i: (0, 0)),
            ],
            out_specs=pl.BlockSpec((_ABLK, 128), lambda i: (i, 0)),
            out_shape=jax.ShapeDtypeStruct((_SH, 128), jnp.float32),
        )(*([edge_attr] * 8), we_a, u, we_u, b_e2)

    return call


_EDGE_PROJ = tuple(_make_edge_proj(h) for h in range(_NHALF))


def _node_proj_body(x_ref, ws_ref, wd_ref, ps_ref, pd_ref):
    x = x_ref[...]
    ps_ref[...] = jnp.dot(x, ws_ref[...], preferred_element_type=jnp.float32)
    pd_ref[...] = jnp.dot(x, wd_ref[...], preferred_element_type=jnp.float32)


def _node_proj(x, we_s, we_d):
    blk = 2000
    return pl.pallas_call(
        _node_proj_body,
        grid=(_N // blk,),
        in_specs=[
            pl.BlockSpec((blk, _D), lambda i: (i, 0)),
            pl.BlockSpec((_D, _DE), lambda i: (0, 0)),
            pl.BlockSpec((_D, _DE), lambda i: (0, 0)),
        ],
        out_specs=[
            pl.BlockSpec((blk, _DE), lambda i: (i, 0)),
            pl.BlockSpec((blk, _DE), lambda i: (i, 0)),
        ],
        out_shape=[
            jax.ShapeDtypeStruct((_N, _DE), jnp.float32),
            jax.ShapeDtypeStruct((_N, _DE), jnp.float32),
        ],
    )(x, we_s, we_d)


_MESH = plsc.VectorSubcoreMesh(
    core_axis_name="c", subcore_axis_name="s", num_cores=_NC, num_subcores=_NS
)


def _make_edge_sc(half):
    edge0 = half * _EH

    @functools.partial(
        pl.kernel,
        out_type=jax.ShapeDtypeStruct((2, _NPAD, _DE), jnp.float32),
        mesh=_MESH,
        scratch_types=(
            [pltpu.VMEM_SHARED((_NPAD, _DE), jnp.float32)]
            + [pltpu.VMEM((_CHUNK,), jnp.int32) for _ in range(2 * _NBUF)]
            + [pltpu.VMEM((_CHUNK, _DE), jnp.float32) for _ in range(3 * _NBUF)]
            + [pltpu.VMEM((_TAIL,), jnp.int32) for _ in range(2)]
            + [pltpu.VMEM((_TAIL, _DE), jnp.float32) for _ in range(3)]
            + [pltpu.SemaphoreType.DMA for _ in range(3 * _NBUF)]
        ),
        compiler_params=pltpu.CompilerParams(use_tc_tiling_on_sc=False),
        name=f"edge_sc_h{half}",
    )
    def edge_sc(a_hbm, src_hbm, dst_hbm, ps_hbm, pd_hbm, zero_hbm, out_hbm,
                ebar_sh,
                si0, si1, si2, di0, di1, di2,
                ar0, ar1, ar2, psr0, psr1, psr2, pdr0, pdr1, pdr2,
                tsi, tdi, tar, tps, tpd,
                lin0, lin1, lin2, g0, g1, g2, sc0, sc1, sc2):
        c = lax.axis_index("c")
        s = lax.axis_index("s")
        wid = s * _NC + c

        pltpu.sync_copy(zero_hbm.at[pl.ds(s * _PER, _PER)],
                        ebar_sh.at[pl.ds(s * _PER, _PER)])
        plsc.subcore_barrier()

        base0 = edge0 + wid * _EPW
        acol = (wid // 4) * _DE
        airow = (wid % 4) * _EPW
        si = (si0, si1, si2)
        di = (di0, di1, di2)
        ar = (ar0, ar1, ar2)
        psr = (psr0, psr1, psr2)
        pdr = (pdr0, pdr1, pdr2)
        lin = (lin0, lin1, lin2)
        gse = (g0, g1, g2)
        sce = (sc0, sc1, sc2)

        def lin_cps(j, b):
            base = base0 + j * _CHUNK
            return (
                pltpu.make_async_copy(src_hbm.at[pl.ds(base, _CHUNK)], si[b],
                                      lin[b]),
                pltpu.make_async_copy(dst_hbm.at[pl.ds(base, _CHUNK)], di[b],
                                      lin[b]),
                pltpu.make_async_copy(
                    a_hbm.at[pl.ds(airow + j * _CHUNK, _CHUNK),
                             pl.ds(acol, _DE)],
                    ar[b], lin[b]),
            )

        def g_cps(b):
            return (
                pltpu.make_async_copy(ps_hbm.at[si[b]], psr[b], gse[b]),
                pltpu.make_async_copy(pd_hbm.at[di[b]], pdr[b], gse[b]),
            )

        def issue_lin(j, b):
            for cp in lin_cps(j, b):
                cp.start()

        def wait_lin(j, b):
            for cp in lin_cps(j, b):
                cp.wait()

        def issue_g(b):
            for cp in g_cps(b):
                cp.start()

        def wait_g(b):
            for cp in g_cps(b):
                cp.wait()

        def issue_sc(b):
            pltpu.async_copy(ar[b], ebar_sh.at[di[b]], sce[b], add=True)

        def wait_sc(b):
            pltpu.make_async_copy(ar[b], ebar_sh.at[di[b]], sce[b]).wait()

        def compute(b):
            a, p, q = ar[b], psr[b], pdr[b]
            for r in range(_CHUNK):
                a[r] = jnp.maximum(a[r] + p[r] + q[r], 0.0)

        def step(j, b, b1, b2, g_next=True, lin_next=True, wait_prev=True):
            if g_next:
                wait_lin(j + 1, b1)
                issue_g(b1)
            wait_g(b)
            compute(b)
            if wait_prev:
                wait_sc(b2)
            issue_sc(b)
            if lin_next:
                issue_lin(j + 2, b2)

        issue_lin(0, 0)
        wait_lin(0, 0)
        issue_g(0)
        issue_lin(1, 1)
        step(0, 0, 1, 2, wait_prev=False)
        step(1, 1, 2, 0)
        step(2, 2, 0, 1)

        def group(jj, carry):
            j = 3 * jj
            step(j + 0, 0, 1, 2)
            step(j + 1, 1, 2, 0)
            step(j + 2, 2, 0, 1)
            return carry

        lax.fori_loop(1, _NCH // _NBUF - 1, group, 0)

        step(_NCH - 3, 0, 1, 2)
        step(_NCH - 2, 1, 2, 0, lin_next=False)
        step(_NCH - 1, 2, 0, 1, g_next=False, lin_next=False)
        wait_sc(2)

        tbase = base0 + _NCH * _CHUNK
        pltpu.sync_copy(src_hbm.at[pl.ds(tbase, _TAIL)], tsi)
        pltpu.sync_copy(dst_hbm.at[pl.ds(tbase, _TAIL)], tdi)
        pltpu.sync_copy(
            a_hbm.at[pl.ds(airow + _NCH * _CHUNK, _TAIL), pl.ds(acol, _DE)],
            tar)
        pltpu.async_copy(ps_hbm.at[tsi], tps, g0).wait()
        pltpu.async_copy(pd_hbm.at[tdi], tpd, g1).wait()
        for r in range(_TAIL):
            tar[r] = jnp.maximum(tar[r] + tps[r] + tpd[r], 0.0)
        pltpu.sync_copy(tar, ebar_sh.at[tdi], add=True)

        plsc.subcore_barrier()
        pltpu.sync_copy(ebar_sh.at[pl.ds(s * _PER, _PER)],
                        out_hbm.at[c].at[pl.ds(s * _PER, _PER)])

    return edge_sc


_EDGE_SC = tuple(_make_edge_sc(h) for h in range(_NHALF))


def _node_update_body(eb0_ref, eb1_ref, x_ref, u_ref, wve_ref, wvx_ref,
                      wvu_ref, bv_ref, wue_ref, wuv_ref, wuu_ref, bu_ref,
                      v_ref, uo_ref, esum, vsum):
    j = pl.program_id(0)
    nb = pl.num_programs(0)

    eb = eb0_ref[0] + eb0_ref[1] + eb1_ref[0] + eb1_ref[1]
    acc = jnp.dot(eb, wve_ref[...], preferred_element_type=jnp.float32)
    acc = acc + jnp.dot(x_ref[...], wvx_ref[...],
                        preferred_element_type=jnp.float32)
    acc = acc + jnp.dot(u_ref[...], wvu_ref[...],
                        preferred_element_type=jnp.float32)
    acc = acc + bv_ref[...]
    v = jnp.maximum(acc, 0.0)
    v_ref[...] = v

    @pl.when(j == 0)
    def _():
        esum[...] = jnp.zeros_like(esum)
        vsum[...] = jnp.zeros_like(vsum)

    esum[...] += jnp.sum(eb, axis=0, keepdims=True)
    vsum[...] += jnp.sum(v, axis=0, keepdims=True)

    @pl.when(j == nb - 1)
    def _():
        un = jnp.dot(esum[...] * (1.0 / _E), wue_ref[...],
                     preferred_element_type=jnp.float32)
        un = un + jnp.dot(vsum[...] * (1.0 / _N), wuv_ref[...],
                          preferred_element_type=jnp.float32)
        un = un + jnp.dot(u_ref[...], wuu_ref[...],
                          preferred_element_type=jnp.float32)
        uo_ref[...] = un + bu_ref[...]


def _node_update(eb0, eb1, x, u, wv_e, wv_x, wv_u, bv2, wu_e, wu_v, wu_u,
                 bu2):
    blk = 2000
    return pl.pallas_call(
        _node_update_body,
        grid=(_N // blk,),
        in_specs=[
            pl.BlockSpec((2, blk, _DE), lambda i: (0, i, 0)),
            pl.BlockSpec((2, blk, _DE), lambda i: (0, i, 0)),
            pl.BlockSpec((blk, _D), lambda i: (i, 0)),
            pl.BlockSpec((1, _DU), lambda i: (0, 0)),
            pl.BlockSpec((_DE, _D), lambda i: (0, 0)),
            pl.BlockSpec((_D, _D), lambda i: (0, 0)),
            pl.BlockSpec((_DU, _D), lambda i: (0, 0)),
            pl.BlockSpec((1, _D), lambda i: (0, 0)),
            pl.BlockSpec((_DE, _DU), lambda i: (0, 0)),
            pl.BlockSpec((_D, _DU), lambda i: (0, 0)),
            pl.BlockSpec((_DU, _DU), lambda i: (0, 0)),
            pl.BlockSpec((1, _DU), lambda i: (0, 0)),
        ],
        out_specs=[
            pl.BlockSpec((blk, _D), lambda i: (i, 0)),
            pl.BlockSpec((1, _DU), lambda i: (0, 0)),
        ],
        out_shape=[
            jax.ShapeDtypeStruct((_N, _D), jnp.float32),
            jax.ShapeDtypeStruct((1, _DU), jnp.float32),
        ],
        scratch_shapes=[
            pltpu.VMEM((1, _DE), jnp.float32),
            pltpu.VMEM((1, _D), jnp.float32),
        ],
    )(eb0, eb1, x, u, wv_e, wv_x, wv_u, bv2, wu_e, wu_v, wu_u, bu2)


def kernel(x, edge_index, edge_attr, u, W_e, b_e, W_v, b_v, W_u, b_u):
    x = x.astype(jnp.float32)
    src = edge_index[0].astype(jnp.int32)
    dst = edge_index[1].astype(jnp.int32)

    we_a = W_e[0:_DE]
    we_s = W_e[_DE:_DE + _D]
    we_d = W_e[_DE + _D:_DE + 2 * _D]
    we_u = W_e[_DE + 2 * _D:]

    ps, pd = _node_proj(x, we_s, we_d)
    zeros = jnp.zeros((_NPAD, _DE), jnp.float32)
    b_e2 = b_e.reshape(1, _DE)

    ebar = []
    for h in range(_NHALF):
        ea_h = lax.slice_in_dim(edge_attr, h * _EH, (h + 1) * _EH, axis=0)
        a_h = _EDGE_PROJ[h](ea_h, we_a, u, we_u, b_e2)
        ebar.append(_EDGE_SC[h](a_h, src, dst, ps, pd, zeros))

    wv_e = W_v[0:_DE]
    wv_x = W_v[_DE:_DE + _D]
    wv_u = W_v[_DE + _D:]
    wu_e = W_u[0:_DE]
    wu_v = W_u[_DE:_DE + _D]
    wu_u = W_u[_DE + _D:]

    v_new, u_new = _node_update(
        ebar[0], ebar[1], x, u, wv_e, wv_x, wv_u, b_v.reshape(1, _D),
        wu_e, wu_v, wu_u, b_u.reshape(1, _DU))
    return v_new, u_new
